# Initial kernel scaffold; baseline (speedup 1.0000x reference)
import jax
import jax.numpy as jnp
from jax.experimental import pallas as pl


def kernel(embeds, centroids):
    raise NotImplementedError("write your pallas kernel here")


# speedup vs baseline: 34.1440x; 34.1440x over previous
import jax
import jax.numpy as jnp
from jax.experimental import pallas as pl

_B, _N, _D = 8, 1024, 768
_P = 2048
_TN = 256
_ROWS = _B * _N
_GRID = _ROWS // _TN


def _knn_body(x_ref, c_ref, o_ref):
    x = x_ref[...]
    c = c_ref[...]
    cx = jax.lax.dot_general(
        c, x, (((1,), (1,)), ((), ())), preferred_element_type=jnp.float32
    )
    cn = jnp.sum(c * c, axis=1, keepdims=True)
    rn = jnp.sum(x * x, axis=1)
    d2 = cn + rn[None, :] - 2.0 * cx

    iota = jax.lax.broadcasted_iota(jnp.int32, d2.shape, 0)
    inf = jnp.float32(jnp.inf)
    m1 = jnp.min(d2, axis=0, keepdims=True)
    i1 = jnp.min(jnp.where(d2 == m1, iota, _P), axis=0, keepdims=True)
    d2b = jnp.where(iota == i1, inf, d2)
    m2 = jnp.min(d2b, axis=0, keepdims=True)
    i2 = jnp.min(jnp.where(d2b == m2, iota, _P), axis=0, keepdims=True)
    d2c = jnp.where(iota == i2, inf, d2b)
    m3 = jnp.min(d2c, axis=0, keepdims=True)

    s1 = jnp.sqrt(m1)
    s2 = jnp.sqrt(m2)
    s3 = jnp.sqrt(m3)
    denom = 1.0 + jnp.exp(s1 - s2) + jnp.exp(s1 - s3)
    o_ref[0] = s1 / denom


def kernel(embeds, centroids):
    x = embeds.reshape(_ROWS, _D)
    out = pl.pallas_call(
        _knn_body,
        grid=(_GRID,),
        in_specs=[
            pl.BlockSpec((_TN, _D), lambda g: (g, 0)),
            pl.BlockSpec((_P, _D), lambda g: (0, 0)),
        ],
        out_specs=pl.BlockSpec((1, 1, _TN), lambda g: (g, 0, 0)),
        out_shape=jax.ShapeDtypeStruct((_GRID, 1, _TN), jnp.float32),
    )(x, centroids)
    return out.reshape(_B, 1, 32, 32)
